# baseline (speedup 1.0000x reference)
import functools

import jax
import jax.numpy as jnp
from jax import lax
from jax.experimental import pallas as pl
from jax.experimental.pallas import tpu as pltpu
from jax.experimental.pallas import tpu_sc as plsc

_NC, _NS = 2, 16
_NW = _NC * _NS


def _transpose_table(t, v):
    d = t.shape[0]
    nfull = v // 128
    nvb = nfull + (1 if v % 128 else 0)
    rout = v * d // 128
    nmain = nvb // _NW
    mesh = plsc.VectorSubcoreMesh(core_axis_name="c", subcore_axis_name="s")

    @functools.partial(
        pl.kernel,
        out_type=jax.ShapeDtypeStruct((rout, 128), jnp.float32),
        mesh=mesh,
        scratch_types=[
            [pltpu.VMEM((d, 128), jnp.float32) for _ in range(2)],
            [pltpu.VMEM((32, 128), jnp.float32) for _ in range(2)],
            [pltpu.SemaphoreType.DMA for _ in range(2)],
            [pltpu.SemaphoreType.DMA for _ in range(2)],
        ],
        compiler_params=pltpu.CompilerParams(
            use_tc_tiling_on_sc=True, needs_layout_passes=False
        ),
    )
    def tk(t_hbm, o_hbm, ins, outs, isems, osems):
        wid = lax.axis_index("s") * _NC + lax.axis_index("c")
        iota = lax.iota(jnp.int32, 16)

        def in_slice(vb):
            return t_hbm.at[:, pl.ds(vb * 128, 128)]

        def out_slice(vb):
            return o_hbm.at[pl.ds(vb * 32, 32)]

        def shuffle(in_v, out_v, nr):
            def rbody(r, carry):
                for c16 in range(8):
                    idx_d = iota + 16 * (c16 % 2)
                    idx_c = jnp.full((16,), r * 4 + c16 // 2, jnp.int32)
                    out_v[r, pl.ds(c16 * 16, 16)] = plsc.load_gather(
                        in_v, [idx_d, idx_c]
                    )
                return carry

            lax.fori_loop(0, nr, rbody, 0)

        for p in range(2):
            pltpu.async_copy(in_slice(wid + p * _NW), ins[p], isems[p])

        def group(g, carry):
            for p in range(2):
                j = g * 2 + p
                vb = wid + j * _NW
                pltpu.make_async_copy(in_slice(vb), ins[p], isems[p]).wait()

                @pl.when(j >= 2)
                def _():
                    pltpu.make_async_copy(
                        outs[p], out_slice(wid + (j - 2) * _NW), osems[p]
                    ).wait()

                shuffle(ins[p], outs[p], 32)
                pltpu.async_copy(outs[p], out_slice(vb), osems[p])

                @pl.when(j + 2 < nmain)
                def _():
                    pltpu.async_copy(
                        in_slice(wid + (j + 2) * _NW), ins[p], isems[p]
                    )

            return carry

        lax.fori_loop(0, nmain // 2, group, 0)
        for p in range(2):
            vb = wid + (nmain - 2 + p) * _NW
            pltpu.make_async_copy(outs[p], out_slice(vb), osems[p]).wait()

        vb_e = wid + nmain * _NW

        @pl.when(vb_e < nfull)
        def _():
            pltpu.sync_copy(in_slice(vb_e), ins[0])
            shuffle(ins[0], outs[0], 32)
            pltpu.sync_copy(outs[0], out_slice(vb_e))

        @pl.when(vb_e == nfull)
        def _():
            pltpu.sync_copy(in_slice(vb_e), ins[0])
            shuffle(ins[0], outs[0], (v % 128) * d // 128)
            pltpu.sync_copy(
                outs[0].at[pl.ds(0, (v % 128) * d // 128)],
                o_hbm.at[pl.ds(vb_e * 32, (v % 128) * d // 128)],
            )

    return tk(t)


def _gather_rows(table, idx5, b, f, d):
    nbr = 16
    bpw = b // nbr
    nf2 = f // 2
    mesh = plsc.VectorSubcoreMesh(core_axis_name="c", subcore_axis_name="s")

    @functools.partial(
        pl.kernel,
        out_type=jax.ShapeDtypeStruct((f, 4, b // 128, 8, 128), jnp.float32),
        mesh=mesh,
        scratch_types=[
            [pltpu.VMEM((bpw,), jnp.int32) for _ in range(2)],
            [pltpu.VMEM((bpw, d), jnp.float32) for _ in range(2)],
            [pltpu.VMEM((bpw // 128, 8, 128), jnp.float32) for _ in range(2)],
            [pltpu.SemaphoreType.DMA for _ in range(2)],
            [pltpu.SemaphoreType.DMA for _ in range(2)],
        ],
        compiler_params=pltpu.CompilerParams(
            use_tc_tiling_on_sc=False, needs_layout_passes=False
        ),
    )
    def gk(tab_hbm, idx_hbm, out_hbm, idxs, rows, tiles, gsems, tsems):
        wid = lax.axis_index("s") * _NC + lax.axis_index("c")
        fh = wid % 2
        br = wid // 2
        b0 = br * bpw
        iota = lax.iota(jnp.int32, 16)

        def out_slice(fg, db):
            return out_hbm.at[fg, db, pl.ds(br * (bpw // 128), bpw // 128)]

        def fire(fi, p):
            pltpu.sync_copy(idx_hbm.at[fh * nf2 + fi, pl.ds(b0, bpw)], idxs[p])
            pltpu.async_copy(tab_hbm.at[idxs[p]], rows[p], gsems[p])

        def chunk(fi, p):
            pltpu.make_async_copy(tab_hbm.at[idxs[p]], rows[p], gsems[p]).wait()
            fg = fh * nf2 + fi
            for db in range(4):
                tp = db % 2
                if db >= 2:
                    pltpu.make_async_copy(
                        tiles[tp], out_slice(fg, db), tsems[tp]
                    ).wait()
                else:

                    @pl.when(fi > 0)
                    def _():
                        pltpu.make_async_copy(
                            tiles[tp], out_slice(fg, db), tsems[tp]
                        ).wait()

                def jbody(jj, carry, _db=db, _tp=tp, _p=p):
                    jbl = jj >> 3
                    din = jj & 7
                    idx_c = jnp.full((16,), _db * 8 + din, jnp.int32)
                    for b16 in range(8):
                        idx_r = iota + (jbl * 128 + b16 * 16)
                        tiles[_tp][jbl, din, pl.ds(b16 * 16, 16)] = (
                            plsc.load_gather(rows[_p], [idx_r, idx_c])
                        )
                    return carry

                lax.fori_loop(0, (bpw // 128) * 8, jbody, 0)
                pltpu.async_copy(tiles[tp], out_slice(fg, db), tsems[tp])

        fire(0, 0)
        fire(1, 1)

        def fgroup(g, carry):
            for p in range(2):
                fi = g * 2 + p
                chunk(fi, p)

                @pl.when(fi + 2 < nf2)
                def _():
                    fire(fi + 2, p)

            return carry

        lax.fori_loop(0, (nf2 - 1) // 2, fgroup, 0)
        chunk(nf2 - 1, (nf2 - 1) % 2)
        fg_last = fh * nf2 + (nf2 - 1)
        for tp in range(2):
            pltpu.make_async_copy(
                tiles[tp], out_slice(fg_last, 2 + tp), tsems[tp]
            ).wait()

    return gk(table, idx5)


def kernel(indices, embedding):
    b, f = indices.shape
    v, s, d = embedding.shape
    sd = s * d
    t = jnp.transpose(embedding.reshape(v, sd), (1, 0))
    rows128 = _transpose_table(t, v)
    table = rows128.reshape(v, sd)
    idx5 = jnp.transpose(indices.astype(jnp.int32), (1, 0))
    x = _gather_rows(table, idx5, b, f, sd)
    out = x.transpose(2, 4, 0, 1, 3).reshape(b, f, s, d)
    return out


# speedup vs baseline: 1.4105x; 1.4105x over previous
import functools

import jax
import jax.numpy as jnp
from jax import lax
from jax.experimental import pallas as pl
from jax.experimental.pallas import tpu as pltpu
from jax.experimental.pallas import tpu_sc as plsc

_NC, _NS = 2, 16
_NW = _NC * _NS


def _transpose_table(t, v):
    d = t.shape[0]
    nfull = v // 128
    nvb = nfull + (1 if v % 128 else 0)
    rout = v * d // 128
    nmain = nvb // _NW
    mesh = plsc.VectorSubcoreMesh(core_axis_name="c", subcore_axis_name="s")

    @functools.partial(
        pl.kernel,
        out_type=jax.ShapeDtypeStruct((rout, 128), jnp.float32),
        mesh=mesh,
        scratch_types=[
            [pltpu.VMEM((d, 128), jnp.float32) for _ in range(2)],
            [pltpu.VMEM((32, 128), jnp.float32) for _ in range(2)],
            [pltpu.SemaphoreType.DMA for _ in range(2)],
            [pltpu.SemaphoreType.DMA for _ in range(2)],
        ],
        compiler_params=pltpu.CompilerParams(
            use_tc_tiling_on_sc=True, needs_layout_passes=False
        ),
    )
    def tk(t_hbm, o_hbm, ins, outs, isems, osems):
        wid = lax.axis_index("s") * _NC + lax.axis_index("c")
        iota = lax.iota(jnp.int32, 16)

        def in_slice(vb):
            return t_hbm.at[:, pl.ds(vb * 128, 128)]

        def out_slice(vb):
            return o_hbm.at[pl.ds(vb * 32, 32)]

        idx_d0 = iota
        idx_d1 = iota + 16

        def shuffle(in_v, out_v, nr):
            def rbody(r, carry):
                base = r * 4
                vals = []
                for half in range(4):
                    idx_c = jnp.full((16,), base + half, jnp.int32)
                    vals.append(plsc.load_gather(in_v, [idx_d0, idx_c]))
                    vals.append(plsc.load_gather(in_v, [idx_d1, idx_c]))
                for k in range(8):
                    out_v[r, pl.ds(k * 16, 16)] = vals[k]
                return carry

            lax.fori_loop(0, nr, rbody, 0)

        for p in range(2):
            pltpu.async_copy(in_slice(wid + p * _NW), ins[p], isems[p])

        def group(g, carry):
            for p in range(2):
                j = g * 2 + p
                vb = wid + j * _NW
                pltpu.make_async_copy(in_slice(vb), ins[p], isems[p]).wait()

                @pl.when(j >= 2)
                def _():
                    pltpu.make_async_copy(
                        outs[p], out_slice(wid + (j - 2) * _NW), osems[p]
                    ).wait()

                shuffle(ins[p], outs[p], 32)
                pltpu.async_copy(outs[p], out_slice(vb), osems[p])

                @pl.when(j + 2 < nmain)
                def _():
                    pltpu.async_copy(
                        in_slice(wid + (j + 2) * _NW), ins[p], isems[p]
                    )

            return carry

        lax.fori_loop(0, nmain // 2, group, 0)
        for p in range(2):
            vb = wid + (nmain - 2 + p) * _NW
            pltpu.make_async_copy(outs[p], out_slice(vb), osems[p]).wait()

        vb_e = wid + nmain * _NW

        @pl.when(vb_e < nfull)
        def _():
            pltpu.sync_copy(in_slice(vb_e), ins[0])
            shuffle(ins[0], outs[0], 32)
            pltpu.sync_copy(outs[0], out_slice(vb_e))

        @pl.when(vb_e == nfull)
        def _():
            pltpu.sync_copy(in_slice(vb_e), ins[0])
            shuffle(ins[0], outs[0], (v % 128) * d // 128)
            pltpu.sync_copy(
                outs[0].at[pl.ds(0, (v % 128) * d // 128)],
                o_hbm.at[pl.ds(vb_e * 32, (v % 128) * d // 128)],
            )

    return tk(t)


def _gather_rows(table, idx5, b, f, d):
    nbr = 16
    bpw = b // nbr
    nf2 = f // 2
    mesh = plsc.VectorSubcoreMesh(core_axis_name="c", subcore_axis_name="s")

    @functools.partial(
        pl.kernel,
        out_type=jax.ShapeDtypeStruct((f, 4, b // 128, 8, 128), jnp.float32),
        mesh=mesh,
        scratch_types=[
            [pltpu.VMEM((bpw,), jnp.int32) for _ in range(2)],
            [pltpu.VMEM((bpw, d), jnp.float32) for _ in range(2)],
            [pltpu.VMEM((bpw // 128, 8, 128), jnp.float32) for _ in range(2)],
            [pltpu.SemaphoreType.DMA for _ in range(2)],
            [pltpu.SemaphoreType.DMA for _ in range(2)],
        ],
        compiler_params=pltpu.CompilerParams(
            use_tc_tiling_on_sc=False, needs_layout_passes=False
        ),
    )
    def gk(tab_hbm, idx_hbm, out_hbm, idxs, rows, tiles, gsems, tsems):
        wid = lax.axis_index("s") * _NC + lax.axis_index("c")
        fh = wid % 2
        br = wid // 2
        b0 = br * bpw
        iota = lax.iota(jnp.int32, 16)
        iob = [iota + k * 16 for k in range(8)]

        def out_slice(fg, db):
            return out_hbm.at[fg, db, pl.ds(br * (bpw // 128), bpw // 128)]

        def fire(fi, p):
            pltpu.sync_copy(idx_hbm.at[fh * nf2 + fi, pl.ds(b0, bpw)], idxs[p])
            pltpu.async_copy(tab_hbm.at[idxs[p]], rows[p], gsems[p])

        def chunk(fi, p):
            pltpu.make_async_copy(tab_hbm.at[idxs[p]], rows[p], gsems[p]).wait()
            fg = fh * nf2 + fi
            for db in range(4):
                tp = db % 2
                if db >= 2:
                    pltpu.make_async_copy(
                        tiles[tp], out_slice(fg, db), tsems[tp]
                    ).wait()
                else:

                    @pl.when(fi > 0)
                    def _():
                        pltpu.make_async_copy(
                            tiles[tp], out_slice(fg, db), tsems[tp]
                        ).wait()

                tile_ref = tiles[tp]
                rows_ref = rows[p]
                db_base = db * 8

                def jbody(jj, carry):
                    jbl = jj >> 3
                    din = jj & 7
                    idx_c = jnp.full((16,), db_base + din, jnp.int32)
                    rbase = jbl * 128
                    vals = [
                        plsc.load_gather(rows_ref, [iob[b16] + rbase, idx_c])
                        for b16 in range(8)
                    ]
                    for b16 in range(8):
                        tile_ref[jbl, din, pl.ds(b16 * 16, 16)] = vals[b16]
                    return carry

                lax.fori_loop(0, (bpw // 128) * 8, jbody, 0)
                pltpu.async_copy(tiles[tp], out_slice(fg, db), tsems[tp])

        fire(0, 0)
        fire(1, 1)

        def fgroup(g, carry):
            for p in range(2):
                fi = g * 2 + p
                chunk(fi, p)

                @pl.when(fi + 2 < nf2)
                def _():
                    fire(fi + 2, p)

            return carry

        lax.fori_loop(0, (nf2 - 1) // 2, fgroup, 0)
        chunk(nf2 - 1, (nf2 - 1) % 2)
        fg_last = fh * nf2 + (nf2 - 1)
        for tp in range(2):
            pltpu.make_async_copy(
                tiles[tp], out_slice(fg_last, 2 + tp), tsems[tp]
            ).wait()

    return gk(table, idx5)


def kernel(indices, embedding):
    b, f = indices.shape
    v, s, d = embedding.shape
    sd = s * d
    t = jnp.transpose(embedding.reshape(v, sd), (1, 0))
    rows128 = _transpose_table(t, v)
    table = rows128.reshape(v, sd)
    idx5 = jnp.transpose(indices.astype(jnp.int32), (1, 0))
    x = _gather_rows(table, idx5, b, f, sd)
    out = x.transpose(2, 4, 0, 1, 3).reshape(b, f, s, d)
    return out


# speedup vs baseline: 1.4125x; 1.0014x over previous
import functools

import jax
import jax.numpy as jnp
from jax import lax
from jax.experimental import pallas as pl
from jax.experimental.pallas import tpu as pltpu
from jax.experimental.pallas import tpu_sc as plsc

_NC, _NS = 2, 16
_NW = _NC * _NS


def _transpose_table(t, v):
    d = t.shape[0]
    nfull = v // 128
    nvb = nfull + (1 if v % 128 else 0)
    rout = v * d // 128
    nmain = nvb // _NW
    mesh = plsc.VectorSubcoreMesh(core_axis_name="c", subcore_axis_name="s")

    @functools.partial(
        pl.kernel,
        out_type=jax.ShapeDtypeStruct((rout, 128), jnp.float32),
        mesh=mesh,
        scratch_types=[
            [pltpu.VMEM((d, 129), jnp.float32) for _ in range(2)],
            [pltpu.VMEM((32, 128), jnp.float32) for _ in range(2)],
            [pltpu.SemaphoreType.DMA for _ in range(2)],
            [pltpu.SemaphoreType.DMA for _ in range(2)],
        ],
        compiler_params=pltpu.CompilerParams(
            use_tc_tiling_on_sc=True, needs_layout_passes=False
        ),
    )
    def tk(t_hbm, o_hbm, ins, outs, isems, osems):
        wid = lax.axis_index("s") * _NC + lax.axis_index("c")
        iota = lax.iota(jnp.int32, 16)

        def in_slice(vb):
            return t_hbm.at[:, pl.ds(vb * 128, 128)]

        def in_dst(p):
            return ins[p].at[:, pl.ds(0, 128)]

        def out_slice(vb):
            return o_hbm.at[pl.ds(vb * 32, 32)]

        idx_d0 = iota
        idx_d1 = iota + 16

        def shuffle(in_v, out_v, nr):
            def rbody(r, carry):
                base = r * 4
                vals = []
                for half in range(4):
                    idx_c = jnp.full((16,), base + half, jnp.int32)
                    vals.append(plsc.load_gather(in_v, [idx_d0, idx_c]))
                    vals.append(plsc.load_gather(in_v, [idx_d1, idx_c]))
                for k in range(8):
                    out_v[r, pl.ds(k * 16, 16)] = vals[k]
                return carry

            lax.fori_loop(0, nr, rbody, 0)

        for p in range(2):
            pltpu.async_copy(in_slice(wid + p * _NW), in_dst(p), isems[p])

        def group(g, carry):
            for p in range(2):
                j = g * 2 + p
                vb = wid + j * _NW
                pltpu.make_async_copy(in_slice(vb), in_dst(p), isems[p]).wait()

                @pl.when(j >= 2)
                def _():
                    pltpu.make_async_copy(
                        outs[p], out_slice(wid + (j - 2) * _NW), osems[p]
                    ).wait()

                shuffle(ins[p], outs[p], 32)
                pltpu.async_copy(outs[p], out_slice(vb), osems[p])

                @pl.when(j + 2 < nmain)
                def _():
                    pltpu.async_copy(
                        in_slice(wid + (j + 2) * _NW), in_dst(p), isems[p]
                    )

            return carry

        lax.fori_loop(0, nmain // 2, group, 0)
        for p in range(2):
            vb = wid + (nmain - 2 + p) * _NW
            pltpu.make_async_copy(outs[p], out_slice(vb), osems[p]).wait()

        vb_e = wid + nmain * _NW

        @pl.when(vb_e < nfull)
        def _():
            pltpu.sync_copy(in_slice(vb_e), in_dst(0))
            shuffle(ins[0], outs[0], 32)
            pltpu.sync_copy(outs[0], out_slice(vb_e))

        @pl.when(vb_e == nfull)
        def _():
            pltpu.sync_copy(in_slice(vb_e), in_dst(0))
            shuffle(ins[0], outs[0], (v % 128) * d // 128)
            pltpu.sync_copy(
                outs[0].at[pl.ds(0, (v % 128) * d // 128)],
                o_hbm.at[pl.ds(vb_e * 32, (v % 128) * d // 128)],
            )

    return tk(t)


def _gather_rows(table, idx5, b, f, d):
    nbr = 16
    bpw = b // nbr
    nf2 = f // 2
    mesh = plsc.VectorSubcoreMesh(core_axis_name="c", subcore_axis_name="s")

    @functools.partial(
        pl.kernel,
        out_type=jax.ShapeDtypeStruct((f, 4, b // 128, 8, 128), jnp.float32),
        mesh=mesh,
        scratch_types=[
            [pltpu.VMEM((bpw,), jnp.int32) for _ in range(2)],
            [pltpu.VMEM((bpw, d), jnp.float32) for _ in range(2)],
            [pltpu.VMEM((bpw // 128, 8, 128), jnp.float32) for _ in range(2)],
            [pltpu.SemaphoreType.DMA for _ in range(2)],
            [pltpu.SemaphoreType.DMA for _ in range(2)],
        ],
        compiler_params=pltpu.CompilerParams(
            use_tc_tiling_on_sc=False, needs_layout_passes=False
        ),
    )
    def gk(tab_hbm, idx_hbm, out_hbm, idxs, rows, tiles, gsems, tsems):
        wid = lax.axis_index("s") * _NC + lax.axis_index("c")
        fh = wid % 2
        br = wid // 2
        b0 = br * bpw
        iota = lax.iota(jnp.int32, 16)
        iob = [iota + k * 16 for k in range(8)]

        def out_slice(fg, db):
            return out_hbm.at[fg, db, pl.ds(br * (bpw // 128), bpw // 128)]

        def fire(fi, p):
            pltpu.sync_copy(idx_hbm.at[fh * nf2 + fi, pl.ds(b0, bpw)], idxs[p])
            pltpu.async_copy(tab_hbm.at[idxs[p]], rows[p], gsems[p])

        def chunk(fi, p):
            pltpu.make_async_copy(tab_hbm.at[idxs[p]], rows[p], gsems[p]).wait()
            fg = fh * nf2 + fi
            for db in range(4):
                tp = db % 2
                if db >= 2:
                    pltpu.make_async_copy(
                        tiles[tp], out_slice(fg, db), tsems[tp]
                    ).wait()
                else:

                    @pl.when(fi > 0)
                    def _():
                        pltpu.make_async_copy(
                            tiles[tp], out_slice(fg, db), tsems[tp]
                        ).wait()

                tile_ref = tiles[tp]
                rows_ref = rows[p]
                db_base = db * 8

                def jbody(jj, carry):
                    jbl = jj >> 3
                    din = jj & 7
                    idx_c = jnp.full((16,), db_base + din, jnp.int32)
                    rbase = jbl * 128
                    vals = [
                        plsc.load_gather(rows_ref, [iob[b16] + rbase, idx_c])
                        for b16 in range(8)
                    ]
                    for b16 in range(8):
                        tile_ref[jbl, din, pl.ds(b16 * 16, 16)] = vals[b16]
                    return carry

                lax.fori_loop(0, (bpw // 128) * 8, jbody, 0)
                pltpu.async_copy(tiles[tp], out_slice(fg, db), tsems[tp])

        fire(0, 0)
        fire(1, 1)

        def fgroup(g, carry):
            for p in range(2):
                fi = g * 2 + p
                chunk(fi, p)

                @pl.when(fi + 2 < nf2)
                def _():
                    fire(fi + 2, p)

            return carry

        lax.fori_loop(0, (nf2 - 1) // 2, fgroup, 0)
        chunk(nf2 - 1, (nf2 - 1) % 2)
        fg_last = fh * nf2 + (nf2 - 1)
        for tp in range(2):
            pltpu.make_async_copy(
                tiles[tp], out_slice(fg_last, 2 + tp), tsems[tp]
            ).wait()

    return gk(table, idx5)


def kernel(indices, embedding):
    b, f = indices.shape
    v, s, d = embedding.shape
    sd = s * d
    t = jnp.transpose(embedding.reshape(v, sd), (1, 0))
    rows128 = _transpose_table(t, v)
    table = rows128.reshape(v, sd)
    idx5 = jnp.transpose(indices.astype(jnp.int32), (1, 0))
    x = _gather_rows(table, idx5, b, f, sd)
    out = x.transpose(2, 4, 0, 1, 3).reshape(b, f, s, d)
    return out


# speedup vs baseline: 1.4525x; 1.0283x over previous
import functools

import jax
import jax.numpy as jnp
from jax import lax
from jax.experimental import pallas as pl
from jax.experimental.pallas import tpu as pltpu
from jax.experimental.pallas import tpu_sc as plsc

_NC, _NS = 2, 16
_NW = _NC * _NS


def _transpose_table(t, v):
    d = t.shape[0]
    nfull = v // 128
    nvb = nfull + (1 if v % 128 else 0)
    rout = v * d // 128
    nmain = nvb // _NW
    mesh = plsc.VectorSubcoreMesh(core_axis_name="c", subcore_axis_name="s")

    @functools.partial(
        pl.kernel,
        out_type=jax.ShapeDtypeStruct((rout, 128), jnp.float32),
        mesh=mesh,
        scratch_types=[
            [pltpu.VMEM((d, 129), jnp.float32) for _ in range(2)],
            [pltpu.VMEM((32, 128), jnp.float32) for _ in range(2)],
            [pltpu.SemaphoreType.DMA for _ in range(2)],
            [pltpu.SemaphoreType.DMA for _ in range(2)],
        ],
        compiler_params=pltpu.CompilerParams(
            use_tc_tiling_on_sc=True, needs_layout_passes=False
        ),
    )
    def tk(t_hbm, o_hbm, ins, outs, isems, osems):
        wid = lax.axis_index("s") * _NC + lax.axis_index("c")
        iota = lax.iota(jnp.int32, 16)

        def in_slice(vb):
            return t_hbm.at[:, pl.ds(vb * 128, 128)]

        def in_dst(p):
            return ins[p].at[:, pl.ds(0, 128)]

        def out_slice(vb):
            return o_hbm.at[pl.ds(vb * 32, 32)]

        idx_d0 = iota
        idx_d1 = iota + 16

        def shuffle(in_v, out_v, nr):
            def rbody(r, carry):
                base = r * 4
                vals = []
                for half in range(4):
                    idx_c = jnp.full((16,), base + half, jnp.int32)
                    vals.append(plsc.load_gather(in_v, [idx_d0, idx_c]))
                    vals.append(plsc.load_gather(in_v, [idx_d1, idx_c]))
                for k in range(8):
                    out_v[r, pl.ds(k * 16, 16)] = vals[k]
                return carry

            lax.fori_loop(0, nr, rbody, 0)

        for p in range(2):
            pltpu.async_copy(in_slice(wid + p * _NW), in_dst(p), isems[p])

        def group(g, carry):
            for p in range(2):
                j = g * 2 + p
                vb = wid + j * _NW
                pltpu.make_async_copy(in_slice(vb), in_dst(p), isems[p]).wait()

                @pl.when(j >= 2)
                def _():
                    pltpu.make_async_copy(
                        outs[p], out_slice(wid + (j - 2) * _NW), osems[p]
                    ).wait()

                shuffle(ins[p], outs[p], 32)
                pltpu.async_copy(outs[p], out_slice(vb), osems[p])

                @pl.when(j + 2 < nmain)
                def _():
                    pltpu.async_copy(
                        in_slice(wid + (j + 2) * _NW), in_dst(p), isems[p]
                    )

            return carry

        lax.fori_loop(0, nmain // 2, group, 0)
        for p in range(2):
            vb = wid + (nmain - 2 + p) * _NW
            pltpu.make_async_copy(outs[p], out_slice(vb), osems[p]).wait()

        vb_e = wid + nmain * _NW

        @pl.when(vb_e < nfull)
        def _():
            pltpu.sync_copy(in_slice(vb_e), in_dst(0))
            shuffle(ins[0], outs[0], 32)
            pltpu.sync_copy(outs[0], out_slice(vb_e))

        @pl.when(vb_e == nfull)
        def _():
            pltpu.sync_copy(in_slice(vb_e), in_dst(0))
            shuffle(ins[0], outs[0], (v % 128) * d // 128)
            pltpu.sync_copy(
                outs[0].at[pl.ds(0, (v % 128) * d // 128)],
                o_hbm.at[pl.ds(vb_e * 32, (v % 128) * d // 128)],
            )

    return tk(t)


def _gather_rows(table, idx5, b, f, d):
    nbr = 16
    bpw = b // nbr
    nf2 = f // 2
    mesh = plsc.VectorSubcoreMesh(core_axis_name="c", subcore_axis_name="s")

    @functools.partial(
        pl.kernel,
        out_type=jax.ShapeDtypeStruct((f, 4, b // 128, 8, 128), jnp.float32),
        mesh=mesh,
        scratch_types=[
            [pltpu.VMEM((bpw,), jnp.int32) for _ in range(2)],
            [pltpu.VMEM((bpw, d), jnp.float32) for _ in range(2)],
            [pltpu.VMEM((bpw // 128, 8, 128), jnp.float32) for _ in range(2)],
            [pltpu.SemaphoreType.DMA for _ in range(2)],
            [pltpu.SemaphoreType.DMA for _ in range(2)],
        ],
        compiler_params=pltpu.CompilerParams(
            use_tc_tiling_on_sc=False, needs_layout_passes=False
        ),
    )
    def gk(tab_hbm, idx_hbm, out_hbm, idxs, rows, tiles, gsems, tsems):
        wid = lax.axis_index("s") * _NC + lax.axis_index("c")
        fh = wid % 2
        br = wid // 2
        b0 = br * bpw
        iota = lax.iota(jnp.int32, 16)
        iob = [iota + k * 16 for k in range(8)]

        def out_slice(fg, db):
            return out_hbm.at[fg, db, pl.ds(br * (bpw // 128), bpw // 128)]

        def fire(fi, p):
            pltpu.sync_copy(idx_hbm.at[fh * nf2 + fi, pl.ds(b0, bpw)], idxs[p])
            pltpu.async_copy(tab_hbm.at[idxs[p]], rows[p], gsems[p])

        def chunk(fi, p):
            pltpu.make_async_copy(tab_hbm.at[idxs[p]], rows[p], gsems[p]).wait()
            fg = fh * nf2 + fi
            for db in range(4):
                tp = db % 2
                if db >= 2:
                    pltpu.make_async_copy(
                        tiles[tp], out_slice(fg, db), tsems[tp]
                    ).wait()
                else:

                    @pl.when(fi > 0)
                    def _():
                        pltpu.make_async_copy(
                            tiles[tp], out_slice(fg, db), tsems[tp]
                        ).wait()

                tile_ref = tiles[tp]
                rows_ref = rows[p]
                db_base = db * 8

                def jbody(jj, carry):
                    jbl = jj >> 3
                    din = jj & 7
                    idx_c = jnp.full((16,), db_base + din, jnp.int32)
                    rbase = jbl * 128
                    vals = [
                        plsc.load_gather(rows_ref, [iob[b16] + rbase, idx_c])
                        for b16 in range(8)
                    ]
                    for b16 in range(8):
                        tile_ref[jbl, din, pl.ds(b16 * 16, 16)] = vals[b16]
                    return carry

                lax.fori_loop(0, (bpw // 128) * 8, jbody, 0)
                pltpu.async_copy(tiles[tp], out_slice(fg, db), tsems[tp])

        fire(0, 0)
        fire(1, 1)

        def fgroup(g, carry):
            for p in range(2):
                fi = g * 2 + p
                chunk(fi, p)

                @pl.when(fi + 2 < nf2)
                def _():
                    fire(fi + 2, p)

            return carry

        lax.fori_loop(0, (nf2 - 1) // 2, fgroup, 0)
        chunk(nf2 - 1, (nf2 - 1) % 2)
        fg_last = fh * nf2 + (nf2 - 1)
        for tp in range(2):
            pltpu.make_async_copy(
                tiles[tp], out_slice(fg_last, 2 + tp), tsems[tp]
            ).wait()

    return gk(table, idx5)


def kernel(indices, embedding):
    b, f = indices.shape
    v, s, d = embedding.shape
    sd = s * d
    padded = jnp.pad(embedding.reshape(v, sd), ((0, 0), (0, 128 - sd)))
    table = padded.reshape(v * (128 // sd), sd)
    idx5 = jnp.transpose(indices.astype(jnp.int32), (1, 0)) * (128 // sd)
    x = _gather_rows(table, idx5, b, f, sd)
    out = x.transpose(2, 4, 0, 1, 3).reshape(b, f, s, d)
    return out


# speedup vs baseline: 1.4532x; 1.0005x over previous
import functools

import jax
import jax.numpy as jnp
from jax import lax
from jax.experimental import pallas as pl
from jax.experimental.pallas import tpu as pltpu
from jax.experimental.pallas import tpu_sc as plsc

_NC, _NS = 2, 16
_NW = _NC * _NS


def _transpose_table(t, v):
    d = t.shape[0]
    nfull = v // 128
    nvb = nfull + (1 if v % 128 else 0)
    rout = v * d // 128
    nmain = nvb // _NW
    mesh = plsc.VectorSubcoreMesh(core_axis_name="c", subcore_axis_name="s")

    @functools.partial(
        pl.kernel,
        out_type=jax.ShapeDtypeStruct((rout, 128), jnp.float32),
        mesh=mesh,
        scratch_types=[
            [pltpu.VMEM((d, 129), jnp.float32) for _ in range(2)],
            [pltpu.VMEM((32, 128), jnp.float32) for _ in range(2)],
            [pltpu.SemaphoreType.DMA for _ in range(2)],
            [pltpu.SemaphoreType.DMA for _ in range(2)],
        ],
        compiler_params=pltpu.CompilerParams(
            use_tc_tiling_on_sc=True, needs_layout_passes=False
        ),
    )
    def tk(t_hbm, o_hbm, ins, outs, isems, osems):
        wid = lax.axis_index("s") * _NC + lax.axis_index("c")
        iota = lax.iota(jnp.int32, 16)

        def in_slice(vb):
            return t_hbm.at[:, pl.ds(vb * 128, 128)]

        def in_dst(p):
            return ins[p].at[:, pl.ds(0, 128)]

        def out_slice(vb):
            return o_hbm.at[pl.ds(vb * 32, 32)]

        idx_d0 = iota
        idx_d1 = iota + 16

        def shuffle(in_v, out_v, nr):
            def rbody(r, carry):
                base = r * 4
                vals = []
                for half in range(4):
                    idx_c = jnp.full((16,), base + half, jnp.int32)
                    vals.append(plsc.load_gather(in_v, [idx_d0, idx_c]))
                    vals.append(plsc.load_gather(in_v, [idx_d1, idx_c]))
                for k in range(8):
                    out_v[r, pl.ds(k * 16, 16)] = vals[k]
                return carry

            lax.fori_loop(0, nr, rbody, 0)

        for p in range(2):
            pltpu.async_copy(in_slice(wid + p * _NW), in_dst(p), isems[p])

        def group(g, carry):
            for p in range(2):
                j = g * 2 + p
                vb = wid + j * _NW
                pltpu.make_async_copy(in_slice(vb), in_dst(p), isems[p]).wait()

                @pl.when(j >= 2)
                def _():
                    pltpu.make_async_copy(
                        outs[p], out_slice(wid + (j - 2) * _NW), osems[p]
                    ).wait()

                shuffle(ins[p], outs[p], 32)
                pltpu.async_copy(outs[p], out_slice(vb), osems[p])

                @pl.when(j + 2 < nmain)
                def _():
                    pltpu.async_copy(
                        in_slice(wid + (j + 2) * _NW), in_dst(p), isems[p]
                    )

            return carry

        lax.fori_loop(0, nmain // 2, group, 0)
        for p in range(2):
            vb = wid + (nmain - 2 + p) * _NW
            pltpu.make_async_copy(outs[p], out_slice(vb), osems[p]).wait()

        vb_e = wid + nmain * _NW

        @pl.when(vb_e < nfull)
        def _():
            pltpu.sync_copy(in_slice(vb_e), in_dst(0))
            shuffle(ins[0], outs[0], 32)
            pltpu.sync_copy(outs[0], out_slice(vb_e))

        @pl.when(vb_e == nfull)
        def _():
            pltpu.sync_copy(in_slice(vb_e), in_dst(0))
            shuffle(ins[0], outs[0], (v % 128) * d // 128)
            pltpu.sync_copy(
                outs[0].at[pl.ds(0, (v % 128) * d // 128)],
                o_hbm.at[pl.ds(vb_e * 32, (v % 128) * d // 128)],
            )

    return tk(t)


def _gather_rows(table, idx5, b, f, d):
    nbr = 16
    bpw = b // nbr
    nf2 = f // 2
    mesh = plsc.VectorSubcoreMesh(core_axis_name="c", subcore_axis_name="s")

    @functools.partial(
        pl.kernel,
        out_type=jax.ShapeDtypeStruct((f, 4, b // 128, 8, 128), jnp.float32),
        mesh=mesh,
        scratch_types=[
            [pltpu.VMEM((bpw,), jnp.int32) for _ in range(2)],
            [pltpu.VMEM((bpw, d), jnp.float32) for _ in range(2)],
            [pltpu.VMEM((bpw // 128, 8, 128), jnp.float32) for _ in range(2)],
            [pltpu.SemaphoreType.DMA for _ in range(2)],
            [pltpu.SemaphoreType.DMA for _ in range(2)],
        ],
        compiler_params=pltpu.CompilerParams(
            use_tc_tiling_on_sc=False, needs_layout_passes=False
        ),
    )
    def gk(tab_hbm, idx_hbm, out_hbm, idxs, rows, tiles, gsems, tsems):
        wid = lax.axis_index("s") * _NC + lax.axis_index("c")
        fh = wid % 2
        br = wid // 2
        b0 = br * bpw
        iota = lax.iota(jnp.int32, 16)
        iob = [iota + k * 16 for k in range(8)]

        def out_slice(fg, db):
            return out_hbm.at[fg, db, pl.ds(br * (bpw // 128), bpw // 128)]

        def fire(fi, p):
            pltpu.sync_copy(idx_hbm.at[fh * nf2 + fi, pl.ds(b0, bpw)], idxs[p])
            pltpu.async_copy(tab_hbm.at[idxs[p]], rows[p], gsems[p])

        def chunk(fi, p):
            pltpu.make_async_copy(tab_hbm.at[idxs[p]], rows[p], gsems[p]).wait()
            fg = fh * nf2 + fi
            for db in range(4):
                tp = db % 2
                if db >= 2:
                    pltpu.make_async_copy(
                        tiles[tp], out_slice(fg, db), tsems[tp]
                    ).wait()
                else:

                    @pl.when(fi > 0)
                    def _():
                        pltpu.make_async_copy(
                            tiles[tp], out_slice(fg, db), tsems[tp]
                        ).wait()

                tile_ref = tiles[tp]
                rows_ref = rows[p]
                db_base = db * 8

                def jbody(jj, carry):
                    jbl = jj >> 3
                    din = jj & 7
                    idx_c = jnp.full((16,), db_base + din, jnp.int32)
                    rbase = jbl * 128
                    vals = [
                        plsc.load_gather(rows_ref, [iob[b16] + rbase, idx_c])
                        for b16 in range(8)
                    ]
                    for b16 in range(8):
                        tile_ref[jbl, din, pl.ds(b16 * 16, 16)] = vals[b16]
                    return carry

                lax.fori_loop(0, (bpw // 128) * 8, jbody, 0)
                pltpu.async_copy(tiles[tp], out_slice(fg, db), tsems[tp])

        fire(0, 0)
        fire(1, 1)

        def fgroup(g, carry):
            for p in range(2):
                fi = g * 2 + p
                chunk(fi, p)

                @pl.when(fi + 2 < nf2)
                def _():
                    fire(fi + 2, p)

            return carry

        lax.fori_loop(0, (nf2 - 1) // 2, fgroup, 0)
        chunk(nf2 - 1, (nf2 - 1) % 2)
        fg_last = fh * nf2 + (nf2 - 1)
        for tp in range(2):
            pltpu.make_async_copy(
                tiles[tp], out_slice(fg_last, 2 + tp), tsems[tp]
            ).wait()

    return gk(table, idx5)


def kernel(indices, embedding):
    b, f = indices.shape
    v, s, d = embedding.shape
    sd = s * d
    padded = jnp.concatenate(
        [embedding.reshape(v, sd), jnp.zeros((v, 128 - sd), jnp.float32)], axis=1
    )
    table = padded.reshape(v * (128 // sd), sd)
    idx5 = jnp.transpose(indices.astype(jnp.int32), (1, 0)) * (128 // sd)
    x = _gather_rows(table, idx5, b, f, sd)
    out = x.transpose(2, 4, 0, 1, 3).reshape(b, f, s, d)
    return out
